# baseline (speedup 1.0000x reference)
import functools

import jax
import jax.numpy as jnp
from jax import lax
from jax.experimental import pallas as pl
from jax.experimental.pallas import tpu as pltpu
from jax.experimental.pallas import tpu_sc as plsc

NUM_ENTITIES = 1000000
EMBED_DIM = 64
BATCH = 16384
FIELDS = 26

NC = 2
NS = 16
NW = NC * NS

B_PER_W = BATCH // NW
CB = 16
NCHUNK = B_PER_W // CB
ROWS = CB * FIELDS
NBUF = 3
DELAY = NBUF - 1


@functools.partial(
    pl.kernel,
    out_type=jax.ShapeDtypeStruct((BATCH, FIELDS, EMBED_DIM), jnp.float32),
    mesh=plsc.VectorSubcoreMesh(core_axis_name="c", subcore_axis_name="s"),
    scratch_types=[
        pltpu.VMEM((B_PER_W * FIELDS,), jnp.int32),
        [pltpu.VMEM((ROWS, EMBED_DIM), jnp.float32) for _ in range(NBUF)],
        [pltpu.SemaphoreType.DMA for _ in range(NBUF)],
        [pltpu.SemaphoreType.DMA for _ in range(NBUF)],
    ],
    compiler_params=pltpu.CompilerParams(use_tc_tiling_on_sc=False),
)
def _gather_kernel(idx_hbm, table_hbm, out_hbm, idx_v, rows, gsem, wsem):
    wid = lax.axis_index("s") * NC + lax.axis_index("c")
    base = wid * B_PER_W

    pltpu.sync_copy(idx_hbm.at[pl.ds(base * FIELDS, B_PER_W * FIELDS)], idx_v)

    def start_gather(c):
        s = c % NBUF
        pltpu.make_async_copy(
            table_hbm.at[idx_v.at[pl.ds(c * ROWS, ROWS)]], rows[s], gsem[s]
        ).start()

    def finish_and_writeback(c):
        s = c % NBUF
        pltpu.make_async_copy(
            table_hbm.at[idx_v.at[pl.ds(c * ROWS, ROWS)]], rows[s], gsem[s]
        ).wait()
        for r in range(CB):
            pltpu.make_async_copy(
                rows[s].at[pl.ds(r * FIELDS, FIELDS)],
                out_hbm.at[base + c * CB + r],
                wsem[s],
            ).start()

    def wait_writeback(c):
        s = c % NBUF
        for r in range(CB):
            pltpu.make_async_copy(
                rows[s].at[pl.ds(r * FIELDS, FIELDS)],
                out_hbm.at[base + c * CB + r],
                wsem[s],
            ).wait()

    for c in range(NCHUNK + DELAY):
        if c < NCHUNK:
            if c >= NBUF:
                wait_writeback(c - NBUF)
            start_gather(c)
        if c >= DELAY:
            finish_and_writeback(c - DELAY)
    for c in range(max(NCHUNK - NBUF, 0), NCHUNK):
        wait_writeback(c)


def kernel(indices, entity_embeddings):
    flat_idx = indices.astype(jnp.int32).reshape(BATCH * FIELDS)
    return _gather_kernel(flat_idx, entity_embeddings)


# speedup vs baseline: 1.0455x; 1.0455x over previous
import functools

import jax
import jax.numpy as jnp
from jax import lax
from jax.experimental import pallas as pl
from jax.experimental.pallas import tpu as pltpu
from jax.experimental.pallas import tpu_sc as plsc

NUM_ENTITIES = 1000000
EMBED_DIM = 64
BATCH = 16384
FIELDS = 26

NC = 2
NS = 16
NW = NC * NS

B_PER_W = BATCH // NW
ROWS = B_PER_W
NBUF = 3
DELAY = NBUF - 1


@functools.partial(
    pl.kernel,
    out_type=jax.ShapeDtypeStruct((FIELDS * BATCH, EMBED_DIM), jnp.float32),
    mesh=plsc.VectorSubcoreMesh(core_axis_name="c", subcore_axis_name="s"),
    scratch_types=[
        pltpu.VMEM((FIELDS * ROWS,), jnp.int32),
        [pltpu.VMEM((ROWS, EMBED_DIM), jnp.float32) for _ in range(NBUF)],
        [pltpu.SemaphoreType.DMA for _ in range(NBUF)],
        [pltpu.SemaphoreType.DMA for _ in range(NBUF)],
        pltpu.SemaphoreType.DMA,
    ],
    compiler_params=pltpu.CompilerParams(use_tc_tiling_on_sc=False),
)
def _gather_kernel(idx_hbm, table_hbm, out_hbm, idx_v, rows, gsem, wsem, isem):
    wid = lax.axis_index("s") * NC + lax.axis_index("c")
    b0 = wid * B_PER_W

    for f in range(FIELDS):
        pltpu.make_async_copy(
            idx_hbm.at[pl.ds(f * BATCH + b0, ROWS)],
            idx_v.at[pl.ds(f * ROWS, ROWS)],
            isem,
        ).start()
    for f in range(FIELDS):
        pltpu.make_async_copy(
            idx_hbm.at[pl.ds(f * BATCH + b0, ROWS)],
            idx_v.at[pl.ds(f * ROWS, ROWS)],
            isem,
        ).wait()

    def start_gather(f):
        s = f % NBUF
        pltpu.make_async_copy(
            table_hbm.at[idx_v.at[pl.ds(f * ROWS, ROWS)]], rows[s], gsem[s]
        ).start()

    def finish_and_writeback(f):
        s = f % NBUF
        pltpu.make_async_copy(
            table_hbm.at[idx_v.at[pl.ds(f * ROWS, ROWS)]], rows[s], gsem[s]
        ).wait()
        pltpu.make_async_copy(
            rows[s], out_hbm.at[pl.ds(f * BATCH + b0, ROWS)], wsem[s]
        ).start()

    def wait_writeback(f):
        s = f % NBUF
        pltpu.make_async_copy(
            rows[s], out_hbm.at[pl.ds(f * BATCH + b0, ROWS)], wsem[s]
        ).wait()

    for f in range(FIELDS + DELAY):
        if f < FIELDS:
            if f >= NBUF:
                wait_writeback(f - NBUF)
            start_gather(f)
        if f >= DELAY:
            finish_and_writeback(f - DELAY)
    for f in range(max(FIELDS - NBUF, 0), FIELDS):
        wait_writeback(f)


def kernel(indices, entity_embeddings):
    flat_idx = jnp.transpose(indices).astype(jnp.int32).reshape(FIELDS * BATCH)
    out = _gather_kernel(flat_idx, entity_embeddings)
    return jnp.transpose(out.reshape(FIELDS, BATCH, EMBED_DIM), (1, 0, 2))
